# baseline (speedup 1.0000x reference)
import jax
import jax.numpy as jnp
import numpy as np
from jax.experimental import pallas as pl
from jax.experimental.pallas import tpu as pltpu

_SA_CFG = [(1024, 0.1, 32), (256, 0.2, 32), (64, 0.4, 32), (16, 0.8, 32)]


def _sqdist(a, b):
    return (jnp.sum(a * a, -1)[:, :, None] + jnp.sum(b * b, -1)[:, None, :]
            - 2.0 * jnp.einsum('bnd,bmd->bnm', a, b))


def _fps(xyz, npoint):
    Bx, Nx, _ = xyz.shape
    def body(i, state):
        dist, idxs, far = state
        idxs = idxs.at[:, i].set(far)
        c = jnp.take_along_axis(xyz, far[:, None, None], axis=1)
        d = jnp.sum((xyz - c) ** 2, -1)
        dist = jnp.minimum(dist, d)
        far = jnp.argmax(dist, -1).astype(jnp.int32)
        return (dist, idxs, far)
    state = (jnp.full((Bx, Nx), 1e10, jnp.float32),
             jnp.zeros((Bx, npoint), jnp.int32), jnp.zeros((Bx,), jnp.int32))
    _, idxs, _ = jax.lax.fori_loop(0, npoint, body, state)
    return idxs


def _ball_query(radius, nsample, xyz, new_xyz):
    Bx, Nx, _ = xyz.shape
    Sx = new_xyz.shape[1]
    d2 = _sqdist(new_xyz, xyz)
    ar = jnp.broadcast_to(jnp.arange(Nx, dtype=jnp.int32), (Bx, Sx, Nx))
    idx = jnp.where(d2 <= radius * radius, ar, Nx)
    idx = jnp.sort(idx, axis=-1)[:, :, :nsample]
    first = idx[:, :, :1]
    idx = jnp.where(idx == Nx, first, idx)
    return jnp.minimum(idx, Nx - 1)


def _take_rows(feat, idx):
    Bx, Sx, Kx = idx.shape
    flat = idx.reshape(Bx, Sx * Kx)
    g = jnp.take_along_axis(feat, flat[:, :, None], axis=1)
    return g.reshape(Bx, Sx, Kx, feat.shape[-1])


def _sa_stage(xyz, feat, npoint, radius, nsample, layers):
    sg = jax.lax.stop_gradient
    fidx = _fps(sg(xyz), npoint)
    new_xyz = jnp.take_along_axis(xyz, fidx[:, :, None], axis=1)
    gidx = _ball_query(radius, nsample, sg(xyz), sg(new_xyz))
    x = _take_rows(feat, gidx)
    for (w, g, b) in layers:
        x = jax.nn.relu(jnp.einsum('bskc,oc->bsko', x, w) * g + b)
    return new_xyz, jnp.max(x, axis=2)


def _fp_stage(xyz1, xyz2, feat1, feat2, layers):
    sg = jax.lax.stop_gradient
    d2 = _sqdist(sg(xyz1), sg(xyz2))
    negd, idx = jax.lax.top_k(-d2, 3)
    dist = jnp.maximum(-negd, 1e-10)
    w = 1.0 / dist
    w = w / jnp.sum(w, -1, keepdims=True)
    g = _take_rows(feat2, idx.astype(jnp.int32))
    interp = jnp.sum(g * w[..., None], axis=2)
    x = jnp.concatenate([feat1, interp], axis=-1)
    for (wt, ga, be) in layers:
        x = jax.nn.relu(jnp.einsum('bnc,oc->bno', x, wt) * ga + be)
    return x


def _fc_kernel(x_ref, w_ref, g_ref, b_ref, o_ref):
    x = x_ref[0]
    w = w_ref[...]
    y = jnp.dot(x, w.T, preferred_element_type=jnp.float32)
    o_ref[0] = jax.nn.relu(y * g_ref[...] + b_ref[...])


def _fc_pallas(x, w, g, b):
    Bx, Nx, C = x.shape
    O = w.shape[0]
    out = pl.pallas_call(
        _fc_kernel,
        grid=(Bx,),
        in_specs=[
            pl.BlockSpec((1, Nx, C), lambda i: (i, 0, 0)),
            pl.BlockSpec((O, C), lambda i: (0, 0)),
            pl.BlockSpec((O,), lambda i: (0,)),
            pl.BlockSpec((O,), lambda i: (0,)),
        ],
        out_specs=pl.BlockSpec((1, Nx, O), lambda i: (i, 0, 0)),
        out_shape=jax.ShapeDtypeStruct((Bx, Nx, O), jnp.float32),
    )(x, w, g, b)
    return out


def kernel(xyz, features, params):
    l_xyz = [xyz]
    l_feat = [jnp.transpose(features, (0, 2, 1))]
    for i, (npoint, radius, nsample) in enumerate(_SA_CFG):
        nx, nf = _sa_stage(l_xyz[i], l_feat[i], npoint, radius, nsample,
                           params["sa"][i])
        l_xyz.append(nx)
        l_feat.append(nf)
    for i in range(-1, -5, -1):
        l_feat[i - 1] = _fp_stage(l_xyz[i - 1], l_xyz[i], l_feat[i - 1],
                                  l_feat[i], params["fp"][i])
    x = l_feat[0]
    for (w, g, b) in params["fc"]:
        x = _fc_pallas(x, w, g, b)
    return jnp.transpose(x, (0, 2, 1))


# speedup vs baseline: 1.5717x; 1.5717x over previous
import jax
import jax.numpy as jnp
import numpy as np
from jax.experimental import pallas as pl
from jax.experimental.pallas import tpu as pltpu

_SA_CFG = [(1024, 0.1, 32), (256, 0.2, 32), (64, 0.4, 32), (16, 0.8, 32)]


def _sqdist(a, b):
    return (jnp.sum(a * a, -1)[:, :, None] + jnp.sum(b * b, -1)[:, None, :]
            - 2.0 * jnp.einsum('bnd,bmd->bnm', a, b))


_B = 8
_LEVELS = (1024, 256, 64, 16)


def _fps_one_level(pts, S, out_ref):
    N = pts.shape[2]
    iota = jax.lax.broadcasted_iota(jnp.int32, (_B, N), 1)
    iota_s = jax.lax.broadcasted_iota(jnp.int32, (3, _B, S), 2)

    def body(i, carry):
        dist, far, acc = carry
        onehot = iota == far[:, None]
        sel = jnp.where(onehot[None], pts, 0.0)
        c = jnp.sum(sel, axis=2)
        acc = jnp.where(iota_s == i, c[:, :, None], acc)
        d = jnp.sum((pts - c[:, :, None]) ** 2, axis=0)
        dist = jnp.minimum(dist, d)
        m = jnp.max(dist, axis=1, keepdims=True)
        far = jnp.min(jnp.where(dist >= m, iota, N), axis=1).astype(jnp.int32)
        return dist, far, acc

    dist0 = jnp.full((_B, N), 1e10, jnp.float32)
    far0 = jnp.zeros((_B,), jnp.int32)
    acc0 = jnp.zeros((3, _B, S), jnp.float32)
    _, _, acc = jax.lax.fori_loop(0, S, body, (dist0, far0, acc0))
    out_ref[...] = acc


def _fps_levels_kernel(pts_ref, nx1_ref, nx2_ref, nx3_ref, nx4_ref):
    _fps_one_level(pts_ref[...], _LEVELS[0], nx1_ref)
    _fps_one_level(nx1_ref[...], _LEVELS[1], nx2_ref)
    _fps_one_level(nx2_ref[...], _LEVELS[2], nx3_ref)
    _fps_one_level(nx3_ref[...], _LEVELS[3], nx4_ref)


def _fps_levels(xyz):
    pts = jnp.transpose(xyz, (2, 0, 1))
    outs = pl.pallas_call(
        _fps_levels_kernel,
        out_shape=tuple(jax.ShapeDtypeStruct((3, _B, S), jnp.float32)
                        for S in _LEVELS),
    )(pts)
    return [jnp.transpose(o, (1, 2, 0)) for o in outs]


def _ball_query(radius, nsample, xyz, new_xyz):
    Bx, Nx, _ = xyz.shape
    Sx = new_xyz.shape[1]
    d2 = _sqdist(new_xyz, xyz)
    ar = jnp.broadcast_to(jnp.arange(Nx, dtype=jnp.int32), (Bx, Sx, Nx))
    idx = jnp.where(d2 <= radius * radius, ar, Nx)
    idx = jnp.sort(idx, axis=-1)[:, :, :nsample]
    first = idx[:, :, :1]
    idx = jnp.where(idx == Nx, first, idx)
    return jnp.minimum(idx, Nx - 1)


def _take_rows(feat, idx):
    Bx, Sx, Kx = idx.shape
    flat = idx.reshape(Bx, Sx * Kx)
    g = jnp.take_along_axis(feat, flat[:, :, None], axis=1)
    return g.reshape(Bx, Sx, Kx, feat.shape[-1])


def _sa_stage(xyz, new_xyz, feat, radius, nsample, layers):
    gidx = _ball_query(radius, nsample, xyz, new_xyz)
    x = _take_rows(feat, gidx)
    for (w, g, b) in layers:
        x = jax.nn.relu(jnp.einsum('bskc,oc->bsko', x, w) * g + b)
    return jnp.max(x, axis=2)


def _fp_stage(xyz1, xyz2, feat1, feat2, layers):
    sg = jax.lax.stop_gradient
    d2 = _sqdist(sg(xyz1), sg(xyz2))
    negd, idx = jax.lax.top_k(-d2, 3)
    dist = jnp.maximum(-negd, 1e-10)
    w = 1.0 / dist
    w = w / jnp.sum(w, -1, keepdims=True)
    g = _take_rows(feat2, idx.astype(jnp.int32))
    interp = jnp.sum(g * w[..., None], axis=2)
    x = jnp.concatenate([feat1, interp], axis=-1)
    for (wt, ga, be) in layers:
        x = jax.nn.relu(jnp.einsum('bnc,oc->bno', x, wt) * ga + be)
    return x


def _fc_kernel(x_ref, w_ref, g_ref, b_ref, o_ref):
    x = x_ref[0]
    w = w_ref[...]
    y = jnp.dot(x, w.T, preferred_element_type=jnp.float32)
    o_ref[0] = jax.nn.relu(y * g_ref[...] + b_ref[...])


def _fc_pallas(x, w, g, b):
    Bx, Nx, C = x.shape
    O = w.shape[0]
    out = pl.pallas_call(
        _fc_kernel,
        grid=(Bx,),
        in_specs=[
            pl.BlockSpec((1, Nx, C), lambda i: (i, 0, 0)),
            pl.BlockSpec((O, C), lambda i: (0, 0)),
            pl.BlockSpec((O,), lambda i: (0,)),
            pl.BlockSpec((O,), lambda i: (0,)),
        ],
        out_specs=pl.BlockSpec((1, Nx, O), lambda i: (i, 0, 0)),
        out_shape=jax.ShapeDtypeStruct((Bx, Nx, O), jnp.float32),
    )(x, w, g, b)
    return out


def kernel(xyz, features, params):
    l_xyz = [xyz] + _fps_levels(xyz)
    l_feat = [jnp.transpose(features, (0, 2, 1))]
    for i, (npoint, radius, nsample) in enumerate(_SA_CFG):
        nf = _sa_stage(l_xyz[i], l_xyz[i + 1], l_feat[i], radius, nsample,
                       params["sa"][i])
        l_feat.append(nf)
    for i in range(-1, -5, -1):
        l_feat[i - 1] = _fp_stage(l_xyz[i - 1], l_xyz[i], l_feat[i - 1],
                                  l_feat[i], params["fp"][i])
    x = l_feat[0]
    for (w, g, b) in params["fc"]:
        x = _fc_pallas(x, w, g, b)
    return jnp.transpose(x, (0, 2, 1))


# speedup vs baseline: 2.1295x; 1.3549x over previous
import jax
import jax.numpy as jnp
import numpy as np
from jax.experimental import pallas as pl
from jax.experimental.pallas import tpu as pltpu

_SA_CFG = [(1024, 0.1, 32), (256, 0.2, 32), (64, 0.4, 32), (16, 0.8, 32)]


def _sqdist(a, b):
    return (jnp.sum(a * a, -1)[:, :, None] + jnp.sum(b * b, -1)[:, None, :]
            - 2.0 * jnp.einsum('bnd,bmd->bnm', a, b))


_B = 8
_LEVELS = (1024, 256, 64, 16)


def _fps_one_level(pts, S, out_ref):
    N = pts.shape[2]
    iota = jax.lax.broadcasted_iota(jnp.int32, (_B, N), 1)
    iota_s = jax.lax.broadcasted_iota(jnp.int32, (3, _B, S), 2)

    def body(i, carry):
        dist, far, acc = carry
        onehot = iota == far[:, None]
        sel = jnp.where(onehot[None], pts, 0.0)
        c = jnp.sum(sel, axis=2)
        acc = jnp.where(iota_s == i, c[:, :, None], acc)
        d = jnp.sum((pts - c[:, :, None]) ** 2, axis=0)
        dist = jnp.minimum(dist, d)
        m = jnp.max(dist, axis=1, keepdims=True)
        far = jnp.min(jnp.where(dist >= m, iota, N), axis=1).astype(jnp.int32)
        return dist, far, acc

    dist0 = jnp.full((_B, N), 1e10, jnp.float32)
    far0 = jnp.zeros((_B,), jnp.int32)
    acc0 = jnp.zeros((3, _B, S), jnp.float32)
    _, _, acc = jax.lax.fori_loop(0, S, body, (dist0, far0, acc0))
    out_ref[...] = acc


def _fps_levels_kernel(pts_ref, nx1_ref, nx2_ref, nx3_ref, nx4_ref):
    _fps_one_level(pts_ref[...], _LEVELS[0], nx1_ref)
    _fps_one_level(nx1_ref[...], _LEVELS[1], nx2_ref)
    _fps_one_level(nx2_ref[...], _LEVELS[2], nx3_ref)
    _fps_one_level(nx3_ref[...], _LEVELS[3], nx4_ref)


def _fps_levels(xyz):
    pts = jnp.transpose(xyz, (2, 0, 1))
    outs = pl.pallas_call(
        _fps_levels_kernel,
        out_shape=tuple(jax.ShapeDtypeStruct((3, _B, S), jnp.float32)
                        for S in _LEVELS),
    )(pts)
    return ([jnp.transpose(o, (1, 2, 0)) for o in outs],
            [jnp.transpose(o, (1, 0, 2)) for o in outs])


def _ball_query_kernel(nxt_ref, xyzt_ref, idx_ref, *, radius, nsample):
    a = nxt_ref[0]
    b = xyzt_ref[0]
    TS = a.shape[1]
    N = b.shape[1]
    a2 = jnp.sum(a * a, axis=0)
    b2 = jnp.sum(b * b, axis=0)
    ab = jax.lax.dot_general(a, b, (((0,), (0,)), ((), ())),
                             preferred_element_type=jnp.float32)
    d2 = a2[:, None] + b2[None, :] - 2.0 * ab
    iota = jax.lax.broadcasted_iota(jnp.int32, (TS, N), 1)
    key = jnp.where(d2 <= radius * radius, iota, N)
    iota_k = jax.lax.broadcasted_iota(jnp.int32, (TS, nsample), 1)
    acc = jnp.full((TS, nsample), N, jnp.int32)
    for k in range(nsample):
        m = jnp.min(key, axis=1)
        acc = jnp.where(iota_k == k, m[:, None], acc)
        key = jnp.where(key == m[:, None], N, key)
    first = acc[:, 0:1]
    idx_ref[0] = jnp.where(acc >= N, jnp.broadcast_to(first, acc.shape), acc)


def _ball_query(radius, nsample, xyzt, nxt):
    Bx, _, N = xyzt.shape
    S = nxt.shape[2]
    TS = min(S, 256)
    import functools
    body = functools.partial(_ball_query_kernel, radius=radius,
                             nsample=nsample)
    return pl.pallas_call(
        body,
        grid=(Bx, S // TS),
        in_specs=[
            pl.BlockSpec((1, 3, TS), lambda i, j: (i, 0, j)),
            pl.BlockSpec((1, 3, N), lambda i, j: (i, 0, 0)),
        ],
        out_specs=pl.BlockSpec((1, TS, nsample), lambda i, j: (i, j, 0)),
        out_shape=jax.ShapeDtypeStruct((Bx, S, nsample), jnp.int32),
    )(nxt, xyzt)


def _take_rows(feat, idx):
    Bx, Sx, Kx = idx.shape
    flat = idx.reshape(Bx, Sx * Kx)
    g = jnp.take_along_axis(feat, flat[:, :, None], axis=1)
    return g.reshape(Bx, Sx, Kx, feat.shape[-1])


def _sa_stage(xyzt, nxt, feat, radius, nsample, layers):
    gidx = _ball_query(radius, nsample, xyzt, nxt)
    x = _take_rows(feat, gidx)
    for (w, g, b) in layers:
        x = jax.nn.relu(jnp.einsum('bskc,oc->bsko', x, w) * g + b)
    return jnp.max(x, axis=2)


def _fp_stage(xyz1, xyz2, feat1, feat2, layers):
    sg = jax.lax.stop_gradient
    d2 = _sqdist(sg(xyz1), sg(xyz2))
    negd, idx = jax.lax.top_k(-d2, 3)
    dist = jnp.maximum(-negd, 1e-10)
    w = 1.0 / dist
    w = w / jnp.sum(w, -1, keepdims=True)
    g = _take_rows(feat2, idx.astype(jnp.int32))
    interp = jnp.sum(g * w[..., None], axis=2)
    x = jnp.concatenate([feat1, interp], axis=-1)
    for (wt, ga, be) in layers:
        x = jax.nn.relu(jnp.einsum('bnc,oc->bno', x, wt) * ga + be)
    return x


def _fc_kernel(x_ref, w_ref, g_ref, b_ref, o_ref):
    x = x_ref[0]
    w = w_ref[...]
    y = jnp.dot(x, w.T, preferred_element_type=jnp.float32)
    o_ref[0] = jax.nn.relu(y * g_ref[...] + b_ref[...])


def _fc_pallas(x, w, g, b):
    Bx, Nx, C = x.shape
    O = w.shape[0]
    out = pl.pallas_call(
        _fc_kernel,
        grid=(Bx,),
        in_specs=[
            pl.BlockSpec((1, Nx, C), lambda i: (i, 0, 0)),
            pl.BlockSpec((O, C), lambda i: (0, 0)),
            pl.BlockSpec((O,), lambda i: (0,)),
            pl.BlockSpec((O,), lambda i: (0,)),
        ],
        out_specs=pl.BlockSpec((1, Nx, O), lambda i: (i, 0, 0)),
        out_shape=jax.ShapeDtypeStruct((Bx, Nx, O), jnp.float32),
    )(x, w, g, b)
    return out


def kernel(xyz, features, params):
    nx_list, nxt_list = _fps_levels(xyz)
    l_xyz = [xyz] + nx_list
    l_xyzt = [jnp.transpose(xyz, (0, 2, 1))] + nxt_list
    l_feat = [jnp.transpose(features, (0, 2, 1))]
    for i, (npoint, radius, nsample) in enumerate(_SA_CFG):
        nf = _sa_stage(l_xyzt[i], l_xyzt[i + 1], l_feat[i], radius, nsample,
                       params["sa"][i])
        l_feat.append(nf)
    for i in range(-1, -5, -1):
        l_feat[i - 1] = _fp_stage(l_xyz[i - 1], l_xyz[i], l_feat[i - 1],
                                  l_feat[i], params["fp"][i])
    x = l_feat[0]
    for (w, g, b) in params["fc"]:
        x = _fc_pallas(x, w, g, b)
    return jnp.transpose(x, (0, 2, 1))


# speedup vs baseline: 3.7651x; 1.7681x over previous
import functools

import jax
import jax.numpy as jnp
import numpy as np
from jax.experimental import pallas as pl
from jax.experimental.pallas import tpu as pltpu
from jax.experimental.pallas import tpu_sc as plsc

_SA_CFG = [(1024, 0.1, 32), (256, 0.2, 32), (64, 0.4, 32), (16, 0.8, 32)]


def _sqdist(a, b):
    return (jnp.sum(a * a, -1)[:, :, None] + jnp.sum(b * b, -1)[:, None, :]
            - 2.0 * jnp.einsum('bnd,bmd->bnm', a, b))


_B = 8
_LEVELS = (1024, 256, 64, 16)


def _fps_one_level(pts, S, out_ref):
    N = pts.shape[2]
    iota = jax.lax.broadcasted_iota(jnp.int32, (_B, N), 1)
    iota_s = jax.lax.broadcasted_iota(jnp.int32, (3, _B, S), 2)

    def body(i, carry):
        dist, far, acc = carry
        onehot = iota == far[:, None]
        sel = jnp.where(onehot[None], pts, 0.0)
        c = jnp.sum(sel, axis=2)
        acc = jnp.where(iota_s == i, c[:, :, None], acc)
        d = jnp.sum((pts - c[:, :, None]) ** 2, axis=0)
        dist = jnp.minimum(dist, d)
        m = jnp.max(dist, axis=1, keepdims=True)
        far = jnp.min(jnp.where(dist >= m, iota, N), axis=1).astype(jnp.int32)
        return dist, far, acc

    dist0 = jnp.full((_B, N), 1e10, jnp.float32)
    far0 = jnp.zeros((_B,), jnp.int32)
    acc0 = jnp.zeros((3, _B, S), jnp.float32)
    _, _, acc = jax.lax.fori_loop(0, S, body, (dist0, far0, acc0))
    out_ref[...] = acc


def _fps_levels_kernel(pts_ref, nx1_ref, nx2_ref, nx3_ref, nx4_ref):
    _fps_one_level(pts_ref[...], _LEVELS[0], nx1_ref)
    _fps_one_level(nx1_ref[...], _LEVELS[1], nx2_ref)
    _fps_one_level(nx2_ref[...], _LEVELS[2], nx3_ref)
    _fps_one_level(nx3_ref[...], _LEVELS[3], nx4_ref)


def _fps_levels(xyz):
    pts = jnp.transpose(xyz, (2, 0, 1))
    outs = pl.pallas_call(
        _fps_levels_kernel,
        out_shape=tuple(jax.ShapeDtypeStruct((3, _B, S), jnp.float32)
                        for S in _LEVELS),
    )(pts)
    return ([jnp.transpose(o, (1, 2, 0)) for o in outs],
            [jnp.transpose(o, (1, 0, 2)) for o in outs])


def _ball_query_kernel(nxt_ref, xyzt_ref, idx_ref, *, radius, nsample):
    a = nxt_ref[0]
    b = xyzt_ref[0]
    TS = a.shape[1]
    N = b.shape[1]
    a2 = jnp.sum(a * a, axis=0)
    b2 = jnp.sum(b * b, axis=0)
    ab = jax.lax.dot_general(a, b, (((0,), (0,)), ((), ())),
                             preferred_element_type=jnp.float32)
    d2 = a2[:, None] + b2[None, :] - 2.0 * ab
    iota = jax.lax.broadcasted_iota(jnp.int32, (TS, N), 1)
    key = jnp.where(d2 <= radius * radius, iota, N)
    iota_k = jax.lax.broadcasted_iota(jnp.int32, (TS, nsample), 1)
    acc = jnp.full((TS, nsample), N, jnp.int32)
    for k in range(nsample):
        m = jnp.min(key, axis=1)
        acc = jnp.where(iota_k == k, m[:, None], acc)
        key = jnp.where(key == m[:, None], N, key)
    first = acc[:, 0:1]
    idx_ref[0] = jnp.where(acc >= N, jnp.broadcast_to(first, acc.shape), acc)


def _ball_query(radius, nsample, xyzt, nxt):
    Bx, _, N = xyzt.shape
    S = nxt.shape[2]
    TS = min(S, 256)
    import functools
    body = functools.partial(_ball_query_kernel, radius=radius,
                             nsample=nsample)
    return pl.pallas_call(
        body,
        grid=(Bx, S // TS),
        in_specs=[
            pl.BlockSpec((1, 3, TS), lambda i, j: (i, 0, j)),
            pl.BlockSpec((1, 3, N), lambda i, j: (i, 0, 0)),
        ],
        out_specs=pl.BlockSpec((1, TS, nsample), lambda i, j: (i, j, 0)),
        out_shape=jax.ShapeDtypeStruct((Bx, S, nsample), jnp.int32),
    )(nxt, xyzt)


def _take_rows(feat, idx):
    Bx, Sx, Kx = idx.shape
    flat = idx.reshape(Bx, Sx * Kx)
    g = jnp.take_along_axis(feat, flat[:, :, None], axis=1)
    return g.reshape(Bx, Sx, Kx, feat.shape[-1])


def _mlp_kernel(x_ref, *refs):
    nl = (len(refs) - 1) // 3
    out_ref = refs[-1]
    x = x_ref[0]
    for i in range(nl):
        w = refs[3 * i][...]
        g = refs[3 * i + 1][...]
        b = refs[3 * i + 2][...]
        y = jax.lax.dot_general(x, w, (((1,), (1,)), ((), ())),
                                preferred_element_type=jnp.float32)
        x = jax.nn.relu(y * g[None, :] + b[None, :])
    out_ref[0] = x


def _mlp_pallas(x, layers):
    Bx, N, C = x.shape
    O = layers[-1][0].shape[0]
    in_specs = [pl.BlockSpec((1, N, C), lambda i: (i, 0, 0))]
    args = [x]
    for (w, g, b) in layers:
        o, c = w.shape
        in_specs += [pl.BlockSpec((o, c), lambda i: (0, 0)),
                     pl.BlockSpec((o,), lambda i: (0,)),
                     pl.BlockSpec((o,), lambda i: (0,))]
        args += [w, g, b]
    return pl.pallas_call(
        _mlp_kernel,
        grid=(Bx,),
        in_specs=in_specs,
        out_specs=pl.BlockSpec((1, N, O), lambda i: (i, 0, 0)),
        out_shape=jax.ShapeDtypeStruct((Bx, N, O), jnp.float32),
    )(*args)


_NW = 32


def _gather_max_sc(p_flat, idx2, K, C):
    nch = idx2.shape[1]
    gpc = 128 // K
    cpw = nch * gpc
    M = _NW * cpw
    mesh = plsc.VectorSubcoreMesh(core_axis_name="c", subcore_axis_name="s")

    @functools.partial(
        pl.kernel, mesh=mesh,
        out_type=jax.ShapeDtypeStruct((M, C), jnp.float32),
        scratch_types=[
            pltpu.VMEM((nch, 128), jnp.int32),
            pltpu.VMEM((128, C), jnp.float32),
            pltpu.VMEM((cpw, C), jnp.float32),
            pltpu.SemaphoreType.DMA,
        ],
    )
    def k(p_hbm, idx_hbm, out_hbm, idx_v, rows_v, out_v, sem):
        wid = jax.lax.axis_index("s") * 2 + jax.lax.axis_index("c")
        pltpu.sync_copy(idx_hbm.at[wid], idx_v)

        def chunk(g, _):
            pltpu.async_copy(p_hbm.at[idx_v.at[g]], rows_v, sem).wait()
            for j in range(gpc):
                for cc in range(C // 16):
                    sl = pl.ds(cc * 16, 16)
                    acc = rows_v[j * K, sl]
                    for r in range(1, K):
                        acc = jnp.maximum(acc, rows_v[j * K + r, sl])
                    out_v[g * gpc + j, sl] = acc
            return ()

        jax.lax.fori_loop(0, nch, chunk, (), unroll=False)
        pltpu.sync_copy(out_v, out_hbm.at[pl.ds(wid * cpw, cpw)])

    return k(p_flat, idx2)


def _sa_stage(xyzt, nxt, feat, radius, nsample, layers):
    Bx, N, _C = feat.shape
    S = nxt.shape[2]
    gidx = _ball_query(radius, nsample, xyzt, nxt)
    p = _mlp_pallas(feat, layers)
    C = p.shape[-1]
    flat_idx = (gidx + (jnp.arange(Bx, dtype=jnp.int32) * N)[:, None, None])
    if C < 128:
        p = jnp.pad(p, ((0, 0), (0, 0), (0, 128 - C)))
    parts = max(1, C // 128)
    p2 = p.reshape(Bx * N * parts, 128)
    outs = []
    for q in range(parts):
        idx_q = (flat_idx * parts + q).reshape(_NW, -1, 128)
        outs.append(_gather_max_sc(p2, idx_q, nsample, 128))
    pooled = outs[0] if parts == 1 else jnp.concatenate(outs, axis=-1)
    return pooled.reshape(Bx, S, -1)[:, :, :C]


def _fp_stage(xyz1, xyz2, feat1, feat2, layers):
    sg = jax.lax.stop_gradient
    d2 = _sqdist(sg(xyz1), sg(xyz2))
    negd, idx = jax.lax.top_k(-d2, 3)
    dist = jnp.maximum(-negd, 1e-10)
    w = 1.0 / dist
    w = w / jnp.sum(w, -1, keepdims=True)
    g = _take_rows(feat2, idx.astype(jnp.int32))
    interp = jnp.sum(g * w[..., None], axis=2)
    x = jnp.concatenate([feat1, interp], axis=-1)
    for (wt, ga, be) in layers:
        x = jax.nn.relu(jnp.einsum('bnc,oc->bno', x, wt) * ga + be)
    return x


def _fc_kernel(x_ref, w_ref, g_ref, b_ref, o_ref):
    x = x_ref[0]
    w = w_ref[...]
    y = jnp.dot(x, w.T, preferred_element_type=jnp.float32)
    o_ref[0] = jax.nn.relu(y * g_ref[...] + b_ref[...])


def _fc_pallas(x, w, g, b):
    Bx, Nx, C = x.shape
    O = w.shape[0]
    out = pl.pallas_call(
        _fc_kernel,
        grid=(Bx,),
        in_specs=[
            pl.BlockSpec((1, Nx, C), lambda i: (i, 0, 0)),
            pl.BlockSpec((O, C), lambda i: (0, 0)),
            pl.BlockSpec((O,), lambda i: (0,)),
            pl.BlockSpec((O,), lambda i: (0,)),
        ],
        out_specs=pl.BlockSpec((1, Nx, O), lambda i: (i, 0, 0)),
        out_shape=jax.ShapeDtypeStruct((Bx, Nx, O), jnp.float32),
    )(x, w, g, b)
    return out


def kernel(xyz, features, params):
    nx_list, nxt_list = _fps_levels(xyz)
    l_xyz = [xyz] + nx_list
    l_xyzt = [jnp.transpose(xyz, (0, 2, 1))] + nxt_list
    l_feat = [jnp.transpose(features, (0, 2, 1))]
    for i, (npoint, radius, nsample) in enumerate(_SA_CFG):
        nf = _sa_stage(l_xyzt[i], l_xyzt[i + 1], l_feat[i], radius, nsample,
                       params["sa"][i])
        l_feat.append(nf)
    for i in range(-1, -5, -1):
        l_feat[i - 1] = _fp_stage(l_xyz[i - 1], l_xyz[i], l_feat[i - 1],
                                  l_feat[i], params["fp"][i])
    x = l_feat[0]
    for (w, g, b) in params["fc"]:
        x = _fc_pallas(x, w, g, b)
    return jnp.transpose(x, (0, 2, 1))


# speedup vs baseline: 16.2787x; 4.3236x over previous
import functools

import jax
import jax.numpy as jnp
from jax.experimental import pallas as pl
from jax.experimental.pallas import tpu as pltpu
from jax.experimental.pallas import tpu_sc as plsc

_SA_CFG = [(1024, 0.1, 32), (256, 0.2, 32), (64, 0.4, 32), (16, 0.8, 32)]


_B = 8
_LEVELS = (1024, 256, 64, 16)


def _fps_one_level(pts, S, out_ref):
    N = pts.shape[2]
    iota = jax.lax.broadcasted_iota(jnp.int32, (_B, N), 1)
    iota_s = jax.lax.broadcasted_iota(jnp.int32, (3, _B, S), 2)

    def body(i, carry):
        dist, far, acc = carry
        onehot = iota == far[:, None]
        sel = jnp.where(onehot[None], pts, 0.0)
        c = jnp.sum(sel, axis=2)
        acc = jnp.where(iota_s == i, c[:, :, None], acc)
        d = jnp.sum((pts - c[:, :, None]) ** 2, axis=0)
        dist = jnp.minimum(dist, d)
        m = jnp.max(dist, axis=1, keepdims=True)
        far = jnp.min(jnp.where(dist >= m, iota, N), axis=1).astype(jnp.int32)
        return dist, far, acc

    dist0 = jnp.full((_B, N), 1e10, jnp.float32)
    far0 = jnp.zeros((_B,), jnp.int32)
    acc0 = jnp.zeros((3, _B, S), jnp.float32)
    _, _, acc = jax.lax.fori_loop(0, S, body, (dist0, far0, acc0))
    out_ref[...] = acc


def _fps_levels_kernel(pts_ref, nx1_ref, nx2_ref, nx3_ref, nx4_ref):
    _fps_one_level(pts_ref[...], _LEVELS[0], nx1_ref)
    _fps_one_level(nx1_ref[...], _LEVELS[1], nx2_ref)
    _fps_one_level(nx2_ref[...], _LEVELS[2], nx3_ref)
    _fps_one_level(nx3_ref[...], _LEVELS[3], nx4_ref)


def _fps_levels(xyz):
    pts = jnp.transpose(xyz, (2, 0, 1))
    outs = pl.pallas_call(
        _fps_levels_kernel,
        out_shape=tuple(jax.ShapeDtypeStruct((3, _B, S), jnp.float32)
                        for S in _LEVELS),
    )(pts)
    return ([jnp.transpose(o, (1, 2, 0)) for o in outs],
            [jnp.transpose(o, (1, 0, 2)) for o in outs])


def _ball_query_kernel(nxt_ref, xyzt_ref, idx_ref, *, radius, nsample):
    a = nxt_ref[0]
    b = xyzt_ref[0]
    TS = a.shape[1]
    N = b.shape[1]
    a2 = jnp.sum(a * a, axis=0)
    b2 = jnp.sum(b * b, axis=0)
    ab = jax.lax.dot_general(a, b, (((0,), (0,)), ((), ())),
                             preferred_element_type=jnp.float32)
    d2 = a2[:, None] + b2[None, :] - 2.0 * ab
    iota = jax.lax.broadcasted_iota(jnp.int32, (TS, N), 1)
    key = jnp.where(d2 <= radius * radius, iota, N)
    iota_k = jax.lax.broadcasted_iota(jnp.int32, (TS, nsample), 1)
    acc = jnp.full((TS, nsample), N, jnp.int32)
    for k in range(nsample):
        m = jnp.min(key, axis=1)
        acc = jnp.where(iota_k == k, m[:, None], acc)
        key = jnp.where(key == m[:, None], N, key)
    first = acc[:, 0:1]
    idx_ref[0] = jnp.where(acc >= N, jnp.broadcast_to(first, acc.shape), acc)


def _ball_query(radius, nsample, xyzt, nxt):
    Bx, _, N = xyzt.shape
    S = nxt.shape[2]
    TS = min(S, 256)
    import functools
    body = functools.partial(_ball_query_kernel, radius=radius,
                             nsample=nsample)
    return pl.pallas_call(
        body,
        grid=(Bx, S // TS),
        in_specs=[
            pl.BlockSpec((1, 3, TS), lambda i, j: (i, 0, j)),
            pl.BlockSpec((1, 3, N), lambda i, j: (i, 0, 0)),
        ],
        out_specs=pl.BlockSpec((1, TS, nsample), lambda i, j: (i, j, 0)),
        out_shape=jax.ShapeDtypeStruct((Bx, S, nsample), jnp.int32),
    )(nxt, xyzt)


def _mlp_kernel(x_ref, *refs):
    nl = (len(refs) - 1) // 3
    out_ref = refs[-1]
    x = x_ref[0]
    for i in range(nl):
        w = refs[3 * i][...]
        g = refs[3 * i + 1][...]
        b = refs[3 * i + 2][...]
        y = jax.lax.dot_general(x, w, (((1,), (1,)), ((), ())),
                                preferred_element_type=jnp.float32)
        x = jax.nn.relu(y * g[None, :] + b[None, :])
    out_ref[0] = x


def _mlp_pallas(x, layers):
    Bx, N, C = x.shape
    O = layers[-1][0].shape[0]
    in_specs = [pl.BlockSpec((1, N, C), lambda i: (i, 0, 0))]
    args = [x]
    for (w, g, b) in layers:
        o, c = w.shape
        in_specs += [pl.BlockSpec((o, c), lambda i: (0, 0)),
                     pl.BlockSpec((o,), lambda i: (0,)),
                     pl.BlockSpec((o,), lambda i: (0,))]
        args += [w, g, b]
    return pl.pallas_call(
        _mlp_kernel,
        grid=(Bx,),
        in_specs=in_specs,
        out_specs=pl.BlockSpec((1, N, O), lambda i: (i, 0, 0)),
        out_shape=jax.ShapeDtypeStruct((Bx, N, O), jnp.float32),
    )(*args)


_NW = 32


def _gather_max_sc(p_flat, idx2, K, C):
    nch = idx2.shape[1]
    gpc = 128 // K
    cpw = nch * gpc
    M = _NW * cpw
    mesh = plsc.VectorSubcoreMesh(core_axis_name="c", subcore_axis_name="s")

    @functools.partial(
        pl.kernel, mesh=mesh,
        out_type=jax.ShapeDtypeStruct((M, C), jnp.float32),
        scratch_types=[
            pltpu.VMEM((nch, 128), jnp.int32),
            pltpu.VMEM((128, C), jnp.float32),
            pltpu.VMEM((cpw, C), jnp.float32),
            pltpu.SemaphoreType.DMA,
        ],
    )
    def k(p_hbm, idx_hbm, out_hbm, idx_v, rows_v, out_v, sem):
        wid = jax.lax.axis_index("s") * 2 + jax.lax.axis_index("c")
        pltpu.sync_copy(idx_hbm.at[wid], idx_v)

        def chunk(g, _):
            pltpu.async_copy(p_hbm.at[idx_v.at[g]], rows_v, sem).wait()
            for j in range(gpc):
                for cc in range(C // 16):
                    sl = pl.ds(cc * 16, 16)
                    acc = rows_v[j * K, sl]
                    for r in range(1, K):
                        acc = jnp.maximum(acc, rows_v[j * K + r, sl])
                    out_v[g * gpc + j, sl] = acc
            return ()

        jax.lax.fori_loop(0, nch, chunk, (), unroll=False)
        pltpu.sync_copy(out_v, out_hbm.at[pl.ds(wid * cpw, cpw)])

    return k(p_flat, idx2)


def _sa_stage(xyzt, nxt, feat, radius, nsample, layers):
    Bx, N, _C = feat.shape
    S = nxt.shape[2]
    gidx = _ball_query(radius, nsample, xyzt, nxt)
    p = _mlp_pallas(feat, layers)
    C = p.shape[-1]
    flat_idx = (gidx + (jnp.arange(Bx, dtype=jnp.int32) * N)[:, None, None])
    if C < 128:
        p = jnp.pad(p, ((0, 0), (0, 0), (0, 128 - C)))
    parts = max(1, C // 128)
    p2 = p.reshape(Bx * N * parts, 128)
    outs = []
    for q in range(parts):
        idx_q = (flat_idx * parts + q).reshape(_NW, -1, 128)
        outs.append(_gather_max_sc(p2, idx_q, nsample, 128))
    pooled = outs[0] if parts == 1 else jnp.concatenate(outs, axis=-1)
    return pooled.reshape(Bx, S, -1)[:, :, :C]


def _fp_kernel(x1_ref, x2_ref, f1_ref, f2_ref, wa_ref, wb_ref, g1_ref,
               b1_ref, *rest):
    out_ref = rest[-1]
    a = x1_ref[0]
    b = x2_ref[0]
    TS = a.shape[1]
    n2 = b.shape[1]
    a2 = jnp.sum(a * a, axis=0)
    b2 = jnp.sum(b * b, axis=0)
    ab = jax.lax.dot_general(a, b, (((0,), (0,)), ((), ())),
                             preferred_element_type=jnp.float32)
    d2 = a2[:, None] + b2[None, :] - 2.0 * ab
    iota = jax.lax.broadcasted_iota(jnp.int32, (TS, n2), 1)
    ohs, ds = [], []
    for _ in range(3):
        m = jnp.min(d2, axis=1)
        oh = jnp.logical_and(d2 == m[:, None],
                             iota == jnp.min(jnp.where(d2 == m[:, None],
                                                       iota, n2),
                                             axis=1)[:, None])
        ohs.append(oh)
        ds.append(m)
        d2 = jnp.where(oh, 1e30, d2)
    dist = [jnp.maximum(d, 1e-10) for d in ds]
    w = [1.0 / d for d in dist]
    tot = w[0] + w[1] + w[2]
    wn = [x / tot for x in w]
    A = (wn[0][:, None] * ohs[0].astype(jnp.float32)
         + wn[1][:, None] * ohs[1].astype(jnp.float32)
         + wn[2][:, None] * ohs[2].astype(jnp.float32))
    interp = jnp.dot(A, f2_ref[0], preferred_element_type=jnp.float32)
    y = (jax.lax.dot_general(f1_ref[0], wa_ref[...], (((1,), (1,)), ((), ())),
                             preferred_element_type=jnp.float32)
         + jax.lax.dot_general(interp, wb_ref[...], (((1,), (1,)), ((), ())),
                               preferred_element_type=jnp.float32))
    x = jax.nn.relu(y * g1_ref[...][None, :] + b1_ref[...][None, :])
    nl = (len(rest) - 1) // 3
    for i in range(nl):
        wt = rest[3 * i][...]
        ga = rest[3 * i + 1][...]
        be = rest[3 * i + 2][...]
        yy = jax.lax.dot_general(x, wt, (((1,), (1,)), ((), ())),
                                 preferred_element_type=jnp.float32)
        x = jax.nn.relu(yy * ga[None, :] + be[None, :])
    out_ref[0] = x


def _fp_stage(xyz1t, xyz2t, feat1, feat2, layers, extra_layers=()):
    Bx, _, n1 = xyz1t.shape
    n2 = xyz2t.shape[2]
    C1 = feat1.shape[2]
    C2 = feat2.shape[2]
    all_layers = list(layers) + list(extra_layers)
    (w1, g1, b1) = all_layers[0]
    w1a, w1b = w1[:, :C1], w1[:, C1:]
    TS = min(n1, 512)
    grid = (Bx, n1 // TS)
    in_specs = [
        pl.BlockSpec((1, 3, TS), lambda i, j: (i, 0, j)),
        pl.BlockSpec((1, 3, n2), lambda i, j: (i, 0, 0)),
        pl.BlockSpec((1, TS, C1), lambda i, j: (i, j, 0)),
        pl.BlockSpec((1, n2, C2), lambda i, j: (i, 0, 0)),
        pl.BlockSpec(w1a.shape, lambda i, j: (0, 0)),
        pl.BlockSpec(w1b.shape, lambda i, j: (0, 0)),
        pl.BlockSpec(g1.shape, lambda i, j: (0,)),
        pl.BlockSpec(b1.shape, lambda i, j: (0,)),
    ]
    args = [xyz1t, xyz2t, feat1, feat2, w1a, w1b, g1, b1]
    for (wt, ga, be) in all_layers[1:]:
        in_specs += [pl.BlockSpec(wt.shape, lambda i, j: (0, 0)),
                     pl.BlockSpec(ga.shape, lambda i, j: (0,)),
                     pl.BlockSpec(be.shape, lambda i, j: (0,))]
        args += [wt, ga, be]
    O = all_layers[-1][0].shape[0]
    return pl.pallas_call(
        _fp_kernel,
        grid=grid,
        in_specs=in_specs,
        out_specs=pl.BlockSpec((1, TS, O), lambda i, j: (i, j, 0)),
        out_shape=jax.ShapeDtypeStruct((Bx, n1, O), jnp.float32),
    )(*args)


def kernel(xyz, features, params):
    nx_list, nxt_list = _fps_levels(xyz)
    l_xyz = [xyz] + nx_list
    l_xyzt = [jnp.transpose(xyz, (0, 2, 1))] + nxt_list
    l_feat = [jnp.transpose(features, (0, 2, 1))]
    for i, (npoint, radius, nsample) in enumerate(_SA_CFG):
        nf = _sa_stage(l_xyzt[i], l_xyzt[i + 1], l_feat[i], radius, nsample,
                       params["sa"][i])
        l_feat.append(nf)
    for i in range(-1, -5, -1):
        extra = params["fc"] if i == -4 else ()
        l_feat[i - 1] = _fp_stage(l_xyzt[i - 1], l_xyzt[i], l_feat[i - 1],
                                  l_feat[i], params["fp"][i], extra)
    return jnp.transpose(l_feat[0], (0, 2, 1))
